# baseline (speedup 1.0000x reference)
import jax
import jax.numpy as jnp
from jax import lax
from jax.experimental import pallas as pl
from jax.experimental.pallas import tpu as pltpu, tpu_sc as plsc

_NUM_CORES = 2
_NUM_SUBCORES = 16
_NW = _NUM_CORES * _NUM_SUBCORES

_BATCH = 16384
_DIM = 64
_B_PER_W = _BATCH // _NW
_CHUNK = 128
_NCHUNK = _B_PER_W // _CHUNK


_NBUF = 4


def _body(user_hbm, item_hbm, mfu_hbm, mfi_hbm, mlu_hbm, mli_hbm,
          out_mfu, out_mfi, out_mlp,
          idx_u, idx_i, buf0, buf1, buf2, buf3, sem0, sem1, sem2, sem3):
    wid = lax.axis_index("s") * _NUM_CORES + lax.axis_index("c")
    base = wid * _B_PER_W

    pltpu.sync_copy(user_hbm.at[pl.ds(base, _B_PER_W)], idx_u)
    pltpu.sync_copy(item_hbm.at[pl.ds(base, _B_PER_W)], idx_i)

    bufs = (buf0, buf1, buf2, buf3)
    sems = (sem0, sem1, sem2, sem3)
    jobs = (
        (idx_u, mfu_hbm, lambda lo, n: out_mfu.at[pl.ds(lo, n)]),
        (idx_i, mfi_hbm, lambda lo, n: out_mfi.at[pl.ds(lo, n)]),
        (idx_u, mlu_hbm, lambda lo, n: out_mlp.at[pl.ds(lo, n), 0]),
        (idx_i, mli_hbm, lambda lo, n: out_mlp.at[pl.ds(lo, n), 1]),
    )
    nsteps = len(jobs) * _NCHUNK

    def gather(step):
        b = step % _NBUF
        idx, table, _ = jobs[step // _NCHUNK]
        c = step % _NCHUNK
        chunk_idx = idx.at[pl.ds(c * _CHUNK, _CHUNK)]
        return pltpu.async_copy(table.at[chunk_idx], bufs[b], sems[b])

    def store(step):
        b = step % _NBUF
        _, _, out_at = jobs[step // _NCHUNK]
        c = step % _NCHUNK
        return pltpu.async_copy(bufs[b], out_at(base + c * _CHUNK, _CHUNK),
                                sems[b])

    gh = [None] * _NBUF
    sh = [None] * _NBUF
    for step in range(min(_NBUF, nsteps)):
        gh[step % _NBUF] = gather(step)
    for step in range(nsteps):
        b = step % _NBUF
        gh[b].wait()
        sh[b] = store(step)
        nxt = step + _NBUF
        if nxt < nsteps:
            sh[b].wait()
            gh[b] = gather(nxt)
    for step in range(max(0, nsteps - _NBUF), nsteps):
        sh[step % _NBUF].wait()


@jax.jit
def _run(user_input, item_input, mfu, mfi, mlu, mli):
    mesh = plsc.VectorSubcoreMesh(core_axis_name="c", subcore_axis_name="s")
    fn = pl.kernel(
        _body,
        out_type=(
            jax.ShapeDtypeStruct((_BATCH, _DIM), jnp.float32),
            jax.ShapeDtypeStruct((_BATCH, _DIM), jnp.float32),
            jax.ShapeDtypeStruct((_BATCH, 2, _DIM), jnp.float32),
        ),
        mesh=mesh,
        compiler_params=pltpu.CompilerParams(use_tc_tiling_on_sc=False),
        scratch_types=[
            pltpu.VMEM((_B_PER_W,), jnp.int32),
            pltpu.VMEM((_B_PER_W,), jnp.int32),
            pltpu.VMEM((_CHUNK, _DIM), jnp.float32),
            pltpu.VMEM((_CHUNK, _DIM), jnp.float32),
            pltpu.VMEM((_CHUNK, _DIM), jnp.float32),
            pltpu.VMEM((_CHUNK, _DIM), jnp.float32),
            pltpu.SemaphoreType.DMA,
            pltpu.SemaphoreType.DMA,
            pltpu.SemaphoreType.DMA,
            pltpu.SemaphoreType.DMA,
        ],
    )
    mf_user, mf_item, mlp = fn(user_input, item_input, mfu, mfi, mlu, mli)
    return mf_user, mf_item, mlp.reshape(_BATCH, 2 * _DIM)


def kernel(user_input, item_input, mf_user_table, mf_item_table,
           mlp_user_table, mlp_item_table):
    return _run(user_input.astype(jnp.int32), item_input.astype(jnp.int32),
                mf_user_table, mf_item_table, mlp_user_table, mlp_item_table)


# speedup vs baseline: 1.0010x; 1.0010x over previous
import jax
import jax.numpy as jnp
from jax import lax
from jax.experimental import pallas as pl
from jax.experimental.pallas import tpu as pltpu, tpu_sc as plsc

_NUM_CORES = 2
_NUM_SUBCORES = 16
_NW = _NUM_CORES * _NUM_SUBCORES

_BATCH = 16384
_DIM = 64
_B_PER_W = _BATCH // _NW
_CHUNK = 128
_NCHUNK = _B_PER_W // _CHUNK


_NBUF = 8
_LEAD = 2


def _body(user_hbm, item_hbm, mfu_hbm, mfi_hbm, mlu_hbm, mli_hbm,
          out_mfu, out_mfi, out_mlp,
          idx_u, idx_i, bufs, sems):
    wid = lax.axis_index("s") * _NUM_CORES + lax.axis_index("c")
    base = wid * _B_PER_W

    pltpu.sync_copy(user_hbm.at[pl.ds(base, _B_PER_W)], idx_u)
    pltpu.sync_copy(item_hbm.at[pl.ds(base, _B_PER_W)], idx_i)

    jobs = (
        (idx_u, mfu_hbm, lambda lo, n: out_mfu.at[pl.ds(lo, n)]),
        (idx_i, mfi_hbm, lambda lo, n: out_mfi.at[pl.ds(lo, n)]),
        (idx_u, mlu_hbm, lambda lo, n: out_mlp.at[pl.ds(lo, n), 0]),
        (idx_i, mli_hbm, lambda lo, n: out_mlp.at[pl.ds(lo, n), 1]),
    )
    nsteps = len(jobs) * _NCHUNK

    def gather(step):
        b = step % _NBUF
        idx, table, _ = jobs[step // _NCHUNK]
        c = step % _NCHUNK
        chunk_idx = idx.at[pl.ds(c * _CHUNK, _CHUNK)]
        return pltpu.async_copy(table.at[chunk_idx], bufs[b], sems[b])

    def store(step):
        b = step % _NBUF
        _, _, out_at = jobs[step // _NCHUNK]
        c = step % _NCHUNK
        return pltpu.async_copy(bufs[b], out_at(base + c * _CHUNK, _CHUNK),
                                sems[b])

    gh = [None] * _NBUF
    sh = [None] * _NBUF
    for k in range(min(_LEAD, nsteps)):
        gh[k % _NBUF] = gather(k)
    for k in range(nsteps):
        nxt = k + _LEAD
        if nxt < nsteps:
            b = nxt % _NBUF
            if nxt >= _NBUF:
                sh[b].wait()
            gh[b] = gather(nxt)
        b = k % _NBUF
        gh[b].wait()
        sh[b] = store(k)
    for k in range(max(0, nsteps - _NBUF), nsteps):
        sh[k % _NBUF].wait()


@jax.jit
def _run(user_input, item_input, mfu, mfi, mlu, mli):
    mesh = plsc.VectorSubcoreMesh(core_axis_name="c", subcore_axis_name="s")
    fn = pl.kernel(
        _body,
        out_type=(
            jax.ShapeDtypeStruct((_BATCH, _DIM), jnp.float32),
            jax.ShapeDtypeStruct((_BATCH, _DIM), jnp.float32),
            jax.ShapeDtypeStruct((_BATCH, 2, _DIM), jnp.float32),
        ),
        mesh=mesh,
        compiler_params=pltpu.CompilerParams(use_tc_tiling_on_sc=False),
        scratch_types=[
            pltpu.VMEM((_B_PER_W,), jnp.int32),
            pltpu.VMEM((_B_PER_W,), jnp.int32),
            [pltpu.VMEM((_CHUNK, _DIM), jnp.float32) for _ in range(_NBUF)],
            [pltpu.SemaphoreType.DMA for _ in range(_NBUF)],
        ],
    )
    mf_user, mf_item, mlp = fn(user_input, item_input, mfu, mfi, mlu, mli)
    return mf_user, mf_item, mlp.reshape(_BATCH, 2 * _DIM)


def kernel(user_input, item_input, mf_user_table, mf_item_table,
           mlp_user_table, mlp_item_table):
    return _run(user_input.astype(jnp.int32), item_input.astype(jnp.int32),
                mf_user_table, mf_item_table, mlp_user_table, mlp_item_table)


# speedup vs baseline: 1.1009x; 1.0998x over previous
import jax
import jax.numpy as jnp
from jax import lax
from jax.experimental import pallas as pl
from jax.experimental.pallas import tpu as pltpu, tpu_sc as plsc

_NUM_FEAT = 64
_NUM_ROWS = 100000
_BATCH = 16384
_F_PER_TILE = _NUM_FEAT // 16
_BCHUNK = 4096
_L = 16


def _gather_feature(idx_v, row_v, out_v, c4):
    def body(k, _):
        iv = idx_v[pl.ds(c4 * _BCHUNK + k * _L, _L)]
        out_v[pl.ds(k * _L, _L)] = plsc.load_gather(row_v, [iv])
        return ()
    lax.fori_loop(0, _BCHUNK // _L, body, (), unroll=8)


def _do_tables(s, idx_hbm, tables, outs, idx_v, row_v, obufs, sems):
    pltpu.sync_copy(idx_hbm, idx_v)
    handles = [None, None]
    for t, (table, out) in enumerate(zip(tables, outs)):
        for jf in range(_F_PER_TILE):
            f = s * _F_PER_TILE + jf
            pltpu.sync_copy(table.at[f], row_v)
            for c4 in range(_BATCH // _BCHUNK):
                b = c4 % 2
                if handles[b] is not None:
                    handles[b].wait()
                _gather_feature(idx_v, row_v, obufs[b], c4)
                handles[b] = pltpu.async_copy(
                    obufs[b], out.at[f, pl.ds(c4 * _BCHUNK, _BCHUNK)],
                    sems[b])
    for h in handles:
        if h is not None:
            h.wait()


def _body(user_hbm, item_hbm, mfu_t, mfi_t, mlu_t, mli_t,
          out_mfu, out_mfi, out_mlp,
          idx_v, row_v, obuf0, obuf1, sem0, sem1):
    c = lax.axis_index("c")
    s = lax.axis_index("s")
    obufs = (obuf0, obuf1)
    sems = (sem0, sem1)
    @pl.when(c == 0)
    def _():
        _do_tables(s, user_hbm,
                   (mfu_t, mlu_t),
                   (out_mfu, out_mlp.at[pl.ds(0, _NUM_FEAT)]),
                   idx_v, row_v, obufs, sems)

    @pl.when(c == 1)
    def _():
        _do_tables(s, item_hbm,
                   (mfi_t, mli_t),
                   (out_mfi, out_mlp.at[pl.ds(_NUM_FEAT, _NUM_FEAT)]),
                   idx_v, row_v, obufs, sems)


@jax.jit
def _run(user_input, item_input, mfu, mfi, mlu, mli):
    mesh = plsc.VectorSubcoreMesh(core_axis_name="c", subcore_axis_name="s")
    fn = pl.kernel(
        _body,
        out_type=(
            jax.ShapeDtypeStruct((_NUM_FEAT, _BATCH), jnp.float32),
            jax.ShapeDtypeStruct((_NUM_FEAT, _BATCH), jnp.float32),
            jax.ShapeDtypeStruct((2 * _NUM_FEAT, _BATCH), jnp.float32),
        ),
        mesh=mesh,
        compiler_params=pltpu.CompilerParams(use_tc_tiling_on_sc=False,
                                             needs_layout_passes=False),
        scratch_types=[
            pltpu.VMEM((_BATCH,), jnp.int32),
            pltpu.VMEM((_NUM_ROWS,), jnp.float32),
            pltpu.VMEM((_BCHUNK,), jnp.float32),
            pltpu.VMEM((_BCHUNK,), jnp.float32),
            pltpu.SemaphoreType.DMA,
            pltpu.SemaphoreType.DMA,
        ],
    )
    mfu_t, mfi_t, mlp_t = fn(user_input, item_input,
                             mfu.T, mfi.T, mlu.T, mli.T)
    return mfu_t.T, mfi_t.T, mlp_t.T


def kernel(user_input, item_input, mf_user_table, mf_item_table,
           mlp_user_table, mlp_item_table):
    return _run(user_input.astype(jnp.int32), item_input.astype(jnp.int32),
                mf_user_table, mf_item_table, mlp_user_table, mlp_item_table)


# speedup vs baseline: 2.1832x; 1.9831x over previous
import jax
import jax.numpy as jnp
from jax import lax
from jax.experimental import pallas as pl
from jax.experimental.pallas import tpu as pltpu, tpu_sc as plsc

_NUM_FEAT = 64
_NUM_ROWS = 100000
_BATCH = 16384
_F_PER_TILE = _NUM_FEAT // 16
_BCHUNK = 4096
_L = 16


def _gather_feature(idx_v, row_v, out_v, c4):
    def body(k, _):
        iv = idx_v[pl.ds(c4 * _BCHUNK + k * _L, _L)]
        out_v[pl.ds(k * _L, _L)] = plsc.load_gather(row_v, [iv])
        return ()
    lax.fori_loop(0, _BCHUNK // _L, body, (), unroll=8)


def _do_tables(s, idx_hbm, tables, outs, idx_v, row_v, obufs, sems):
    pltpu.sync_copy(idx_hbm, idx_v)
    handles = [None, None]
    for t, (table, out) in enumerate(zip(tables, outs)):
        for jf in range(_F_PER_TILE):
            f = s * _F_PER_TILE + jf
            pltpu.sync_copy(table.at[f], row_v)
            for c4 in range(_BATCH // _BCHUNK):
                b = c4 % 2
                if handles[b] is not None:
                    handles[b].wait()
                _gather_feature(idx_v, row_v, obufs[b], c4)
                handles[b] = pltpu.async_copy(
                    obufs[b], out.at[f, pl.ds(c4 * _BCHUNK, _BCHUNK)],
                    sems[b])
    for h in handles:
        if h is not None:
            h.wait()


def _body(user_hbm, item_hbm, mfu_t, mfi_t, mlu_t, mli_t,
          out_mfu, out_mfi, out_mlp,
          idx_v, row_v, obuf0, obuf1, sem0, sem1):
    c = lax.axis_index("c")
    s = lax.axis_index("s")
    obufs = (obuf0, obuf1)
    sems = (sem0, sem1)
    @pl.when(c == 0)
    def _():
        _do_tables(s, user_hbm,
                   (mfu_t, mlu_t),
                   (out_mfu, out_mlp.at[pl.ds(0, _NUM_FEAT)]),
                   idx_v, row_v, obufs, sems)

    @pl.when(c == 1)
    def _():
        _do_tables(s, item_hbm,
                   (mfi_t, mli_t),
                   (out_mfi, out_mlp.at[pl.ds(_NUM_FEAT, _NUM_FEAT)]),
                   idx_v, row_v, obufs, sems)


@jax.jit
def _run(user_input, item_input, mfu, mfi, mlu, mli):
    mesh = plsc.VectorSubcoreMesh(core_axis_name="c", subcore_axis_name="s")
    fn = pl.kernel(
        _body,
        out_type=(
            jax.ShapeDtypeStruct((_NUM_FEAT, _BATCH), jnp.float32),
            jax.ShapeDtypeStruct((_NUM_FEAT, _BATCH), jnp.float32),
            jax.ShapeDtypeStruct((2 * _NUM_FEAT, _BATCH), jnp.float32),
        ),
        mesh=mesh,
        compiler_params=pltpu.CompilerParams(use_tc_tiling_on_sc=True,
                                             needs_layout_passes=False),
        scratch_types=[
            pltpu.VMEM((_BATCH,), jnp.int32),
            pltpu.VMEM((_NUM_ROWS,), jnp.float32),
            pltpu.VMEM((_BCHUNK,), jnp.float32),
            pltpu.VMEM((_BCHUNK,), jnp.float32),
            pltpu.SemaphoreType.DMA,
            pltpu.SemaphoreType.DMA,
        ],
    )
    mfu_t, mfi_t, mlp_t = fn(user_input, item_input,
                             mfu.T, mfi.T, mlu.T, mli.T)
    return mfu_t.T, mfi_t.T, mlp_t.T


def kernel(user_input, item_input, mf_user_table, mf_item_table,
           mlp_user_table, mlp_item_table):
    return _run(user_input.astype(jnp.int32), item_input.astype(jnp.int32),
                mf_user_table, mf_item_table, mlp_user_table, mlp_item_table)


# speedup vs baseline: 3.5891x; 1.6439x over previous
import jax
import jax.numpy as jnp
from jax import lax
from jax.experimental import pallas as pl
from jax.experimental.pallas import tpu as pltpu, tpu_sc as plsc

_NUM_FEAT = 64
_NUM_ROWS = 100000
_BATCH = 16384
_F_PER_TILE = _NUM_FEAT // 16
_BCHUNK = 4096
_L = 16


def _gather_feature(idx_v, row_v, out_v, c4):
    @plsc.parallel_loop(0, _BCHUNK, _L, unroll=8)
    def _(k):
        iv = idx_v[pl.ds(c4 * _BCHUNK + k, _L)]
        out_v[pl.ds(k, _L)] = plsc.load_gather(row_v, [iv])


def _do_tables(s, idx_hbm, tables, outs, idx_v, row_v, obufs, sems):
    pltpu.sync_copy(idx_hbm, idx_v)
    handles = [None, None]
    for t, (table, out) in enumerate(zip(tables, outs)):
        for jf in range(_F_PER_TILE):
            f = s * _F_PER_TILE + jf
            pltpu.sync_copy(table.at[f], row_v)
            for c4 in range(_BATCH // _BCHUNK):
                b = c4 % 2
                if handles[b] is not None:
                    handles[b].wait()
                _gather_feature(idx_v, row_v, obufs[b], c4)
                handles[b] = pltpu.async_copy(
                    obufs[b], out.at[f, pl.ds(c4 * _BCHUNK, _BCHUNK)],
                    sems[b])
    for h in handles:
        if h is not None:
            h.wait()


def _body(user_hbm, item_hbm, mfu_t, mfi_t, mlu_t, mli_t,
          out_mfu, out_mfi, out_mlp,
          idx_v, row_v, obuf0, obuf1, sem0, sem1):
    c = lax.axis_index("c")
    s = lax.axis_index("s")
    obufs = (obuf0, obuf1)
    sems = (sem0, sem1)
    @pl.when(c == 0)
    def _():
        _do_tables(s, user_hbm,
                   (mfu_t, mlu_t),
                   (out_mfu, out_mlp.at[pl.ds(0, _NUM_FEAT)]),
                   idx_v, row_v, obufs, sems)

    @pl.when(c == 1)
    def _():
        _do_tables(s, item_hbm,
                   (mfi_t, mli_t),
                   (out_mfi, out_mlp.at[pl.ds(_NUM_FEAT, _NUM_FEAT)]),
                   idx_v, row_v, obufs, sems)


@jax.jit
def _run(user_input, item_input, mfu, mfi, mlu, mli):
    mesh = plsc.VectorSubcoreMesh(core_axis_name="c", subcore_axis_name="s")
    fn = pl.kernel(
        _body,
        out_type=(
            jax.ShapeDtypeStruct((_NUM_FEAT, _BATCH), jnp.float32),
            jax.ShapeDtypeStruct((_NUM_FEAT, _BATCH), jnp.float32),
            jax.ShapeDtypeStruct((2 * _NUM_FEAT, _BATCH), jnp.float32),
        ),
        mesh=mesh,
        compiler_params=pltpu.CompilerParams(use_tc_tiling_on_sc=True,
                                             needs_layout_passes=False),
        scratch_types=[
            pltpu.VMEM((_BATCH,), jnp.int32),
            pltpu.VMEM((_NUM_ROWS,), jnp.float32),
            pltpu.VMEM((_BCHUNK,), jnp.float32),
            pltpu.VMEM((_BCHUNK,), jnp.float32),
            pltpu.SemaphoreType.DMA,
            pltpu.SemaphoreType.DMA,
        ],
    )
    mfu_t, mfi_t, mlp_t = fn(user_input, item_input,
                             mfu.T, mfi.T, mlu.T, mli.T)
    return mfu_t.T, mfi_t.T, mlp_t.T


def kernel(user_input, item_input, mf_user_table, mf_item_table,
           mlp_user_table, mlp_item_table):
    return _run(user_input.astype(jnp.int32), item_input.astype(jnp.int32),
                mf_user_table, mf_item_table, mlp_user_table, mlp_item_table)


# speedup vs baseline: 3.5976x; 1.0024x over previous
import jax
import jax.numpy as jnp
from jax import lax
from jax.experimental import pallas as pl
from jax.experimental.pallas import tpu as pltpu, tpu_sc as plsc

_NUM_FEAT = 64
_NUM_ROWS = 100000
_BATCH = 16384
_F_PER_TILE = _NUM_FEAT // 16
_BCHUNK = 4096
_L = 16


def _gather_feature(idx_v, row_v, out_v, c4):
    @plsc.parallel_loop(0, _BCHUNK, _L, unroll=8)
    def _(k):
        iv = idx_v[pl.ds(c4 * _BCHUNK + k, _L)]
        out_v[pl.ds(k, _L)] = plsc.load_gather(row_v, [iv])


def _do_tables(s, idx_hbm, tables, outs, idx_v, row_v, obufs, sems):
    pltpu.sync_copy(idx_hbm, idx_v)
    handles = [None, None]
    for t, (table, out) in enumerate(zip(tables, outs)):
        for jf in range(_F_PER_TILE):
            f = s * _F_PER_TILE + jf
            pltpu.sync_copy(table.at[f], row_v)
            for c4 in range(_BATCH // _BCHUNK):
                b = c4 % 2
                if handles[b] is not None:
                    handles[b].wait()
                _gather_feature(idx_v, row_v, obufs[b], c4)
                handles[b] = pltpu.async_copy(
                    obufs[b], out.at[f, pl.ds(c4 * _BCHUNK, _BCHUNK)],
                    sems[b])
    for h in handles:
        if h is not None:
            h.wait()


def _body(user_hbm, item_hbm, mfu_t, mfi_t, mlu_t, mli_t,
          out_mfu, out_mfi, out_mlp,
          idx_v, row_v, obuf0, obuf1, sem0, sem1):
    c = lax.axis_index("c")
    s = lax.axis_index("s")
    obufs = (obuf0, obuf1)
    sems = (sem0, sem1)
    @pl.when(c == 0)
    def _():
        _do_tables(s, user_hbm, (mfu_t, mlu_t),
                   (out_mfu, out_mlp.at[pl.ds(0, _NUM_FEAT)]),
                   idx_v, row_v, obufs, sems)

    @pl.when(c == 1)
    def _():
        _do_tables(s, item_hbm, (mfi_t, mli_t),
                   (out_mfi, out_mlp.at[pl.ds(_NUM_FEAT, _NUM_FEAT)]),
                   idx_v, row_v, obufs, sems)


def _run(user_input, item_input, mfu, mfi, mlu, mli):
    mesh = plsc.VectorSubcoreMesh(core_axis_name="c", subcore_axis_name="s")
    fn = pl.kernel(
        _body,
        out_type=(
            jax.ShapeDtypeStruct((_NUM_FEAT, _BATCH), jnp.float32),
            jax.ShapeDtypeStruct((_NUM_FEAT, _BATCH), jnp.float32),
            jax.ShapeDtypeStruct((2 * _NUM_FEAT, _BATCH), jnp.float32),
        ),
        mesh=mesh,
        compiler_params=pltpu.CompilerParams(use_tc_tiling_on_sc=True,
                                             needs_layout_passes=False),
        scratch_types=[
            pltpu.VMEM((_BATCH,), jnp.int32),
            pltpu.VMEM((_NUM_ROWS,), jnp.float32),
            pltpu.VMEM((_BCHUNK,), jnp.float32),
            pltpu.VMEM((_BCHUNK,), jnp.float32),
            pltpu.SemaphoreType.DMA,
            pltpu.SemaphoreType.DMA,
        ],
    )
    mfu_t, mfi_t, mlp_t = fn(user_input, item_input,
                             mfu.T, mfi.T, mlu.T, mli.T)
    return mfu_t.T, mfi_t.T, mlp_t.T


_run_jit = jax.jit(_run)


def kernel(user_input, item_input, mf_user_table, mf_item_table,
           mlp_user_table, mlp_item_table):
    return _run_jit(user_input.astype(jnp.int32),
                    item_input.astype(jnp.int32),
                    mf_user_table, mf_item_table,
                    mlp_user_table, mlp_item_table)
